# baseline (speedup 1.0000x reference)
import functools

import jax
import jax.numpy as jnp
from jax import lax
from jax.experimental import pallas as pl
from jax.experimental.pallas import tpu as pltpu
from jax.experimental.pallas import tpu_sc as plsc

B, T, C = 4, 2048, 768
T_OUT = 2048
BC = 256

TCH = 256
XROWS = TCH + 16
CB = 128
NCB = C // CB
NTC = T // TCH
NTASK = B * NCB * NTC
NW = 32
PER_W = NTASK // NW


def _locnet_body(w_ref, x_ref, b_ref, o_ref, wt_ref):
    b_id = pl.program_id(0)
    c_id = pl.program_id(1)

    @pl.when(jnp.logical_and(b_id == 0, c_id == 0))
    def _():
        wt_ref[...] = w_ref[...].T.astype(jnp.bfloat16)

    x = x_ref[0]
    bias = b_ref[...]
    acc = jax.lax.dot_general(
        wt_ref[...], x.astype(jnp.bfloat16), (((1,), (0,)), ((), ())),
        preferred_element_type=jnp.float32,
    )
    o_ref[0] = jnp.tanh(acc + bias)


def _locnet(X, W_loc, b_loc):
    bias = b_loc.reshape(T_OUT, 1)
    return pl.pallas_call(
        _locnet_body,
        grid=(B, C // BC),
        in_specs=[
            pl.BlockSpec((T, T_OUT), lambda b, c: (0, 0)),
            pl.BlockSpec((1, T, BC), lambda b, c: (b, 0, c)),
            pl.BlockSpec((T_OUT, 1), lambda b, c: (0, 0)),
        ],
        out_specs=pl.BlockSpec((1, T_OUT, BC), lambda b, c: (b, 0, c)),
        out_shape=jax.ShapeDtypeStruct((B, T_OUT, C), jnp.float32),
        scratch_shapes=[pltpu.VMEM((T_OUT, T), jnp.bfloat16)],
        compiler_params=pltpu.CompilerParams(
            dimension_semantics=("arbitrary", "arbitrary"),
        ),
    )(W_loc, X, bias)


@functools.partial(
    pl.kernel,
    mesh=plsc.VectorSubcoreMesh(core_axis_name="c", subcore_axis_name="s"),
    out_type=jax.ShapeDtypeStruct((B, T_OUT, C), jnp.float32),
    scratch_types=[
        pltpu.VMEM((XROWS, CB), jnp.float32),
        pltpu.VMEM((TCH, CB), jnp.float32),
        pltpu.VMEM((TCH, CB), jnp.float32),
    ],
)
def _sc_interp(x_hbm, d_hbm, out_hbm, xbuf, dbuf, obuf):
    wid = lax.axis_index("s") * 2 + lax.axis_index("c")

    def task(k, carry):
        tid = wid * PER_W + k
        tc = tid % NTC
        rest = tid // NTC
        cb = rest % NCB
        b = rest // NCB
        t0 = tc * TCH
        c0 = cb * CB
        src = pl.multiple_of(jnp.clip(t0 - 8, 0, T - XROWS), 8)
        pltpu.sync_copy(x_hbm.at[b, pl.ds(src, XROWS), pl.ds(c0, CB)], xbuf)
        pltpu.sync_copy(d_hbm.at[b, pl.ds(t0, TCH), pl.ds(c0, CB)], dbuf)

        def row(r, carry2):
            o = t0 + r
            i0 = o - src
            im1 = jnp.maximum(o - 1, 0) - src
            ip1 = jnp.minimum(o + 1, T - 1) - src
            for g in range(CB // 16):
                sl = pl.ds(g * 16, 16)
                dv = dbuf[r, sl]
                xm = xbuf[im1, sl]
                x0v = xbuf[i0, sl]
                xp = xbuf[ip1, sl]
                neg = dv < 0.0
                sat = dv >= 1.0
                w1 = jnp.where(neg, dv + 1.0, jnp.where(sat, 0.0, dv))
                w0 = 1.0 - w1
                va = jnp.where(neg, xm, jnp.where(sat, xp, x0v))
                vb = jnp.where(neg, x0v, xp)
                obuf[r, sl] = w0 * va + w1 * vb
            return carry2

        lax.fori_loop(0, TCH, row, 0)

        @pl.when(tc == 0)
        def _():
            for g in range(CB // 16):
                sl = pl.ds(g * 16, 16)
                dv = dbuf[0, sl]
                obuf[0, sl] = jnp.where(dv < 0.0, 0.0, obuf[0, sl])

        @pl.when(tc == NTC - 1)
        def _():
            for g in range(CB // 16):
                sl = pl.ds(g * 16, 16)
                dl = dbuf[TCH - 1, sl]
                obuf[TCH - 1, sl] = jnp.where(dl < 0.0, obuf[TCH - 1, sl], 0.0)
                dp = dbuf[TCH - 2, sl]
                obuf[TCH - 2, sl] = jnp.where(dp >= 1.0, 0.0, obuf[TCH - 2, sl])

        pltpu.sync_copy(obuf, out_hbm.at[b, pl.ds(t0, TCH), pl.ds(c0, CB)])
        return carry

    lax.fori_loop(0, PER_W, task, 0)


@jax.jit
def kernel(X, W_loc, b_loc):
    deform = _locnet(X, W_loc, b_loc)
    return _sc_interp(X, deform)


# speedup vs baseline: 3.5823x; 3.5823x over previous
import functools

import jax
import jax.numpy as jnp
from jax.experimental import pallas as pl
from jax.experimental.pallas import tpu as pltpu

B, T, C = 4, 2048, 768
T_OUT = 2048
BC = 256


def _body(w_ref, x_ref, b_ref, o_ref, wt_ref):
    b_id = pl.program_id(0)
    c_id = pl.program_id(1)

    @pl.when(jnp.logical_and(b_id == 0, c_id == 0))
    def _():
        wt_ref[...] = w_ref[...].T.astype(jnp.bfloat16)

    x = x_ref[0]
    bias = b_ref[...]

    acc = jax.lax.dot_general(
        wt_ref[...], x.astype(jnp.bfloat16), (((1,), (0,)), ((), ())),
        preferred_element_type=jnp.float32,
    )
    d = jnp.tanh(acc + bias)

    neg = d < 0.0
    sat = d >= 1.0
    w1 = d - jnp.floor(d)
    xm1 = jnp.concatenate([x[:1], x[:-1]], axis=0)
    xp1 = jnp.concatenate([x[1:], x[-1:]], axis=0)
    va = jnp.where(neg, xm1, jnp.where(sat, xp1, x))
    vb = jnp.where(neg, x, xp1)
    out = va + w1 * (vb - va)

    zero = jnp.zeros((1, BC), jnp.float32)
    r0 = jnp.where(neg[:1], zero, out[:1])
    rl = jnp.where(neg[-1:], out[-1:], zero)
    rp = jnp.where(sat[T - 2:T - 1], zero, out[T - 2:T - 1])
    o_ref[0] = jnp.concatenate([r0, out[1:T - 2], rp, rl], axis=0)


@jax.jit
def kernel(X, W_loc, b_loc):
    bias = b_loc.reshape(T_OUT, 1)
    grid = (B, C // BC)
    return pl.pallas_call(
        _body,
        grid=grid,
        in_specs=[
            pl.BlockSpec((T, T_OUT), lambda b, c: (0, 0)),
            pl.BlockSpec((1, T, BC), lambda b, c: (b, 0, c)),
            pl.BlockSpec((T_OUT, 1), lambda b, c: (0, 0)),
        ],
        out_specs=pl.BlockSpec((1, T_OUT, BC), lambda b, c: (b, 0, c)),
        out_shape=jax.ShapeDtypeStruct((B, T_OUT, C), jnp.float32),
        scratch_shapes=[pltpu.VMEM((T_OUT, T), jnp.bfloat16)],
        compiler_params=pltpu.CompilerParams(
            dimension_semantics=("arbitrary", "arbitrary"),
        ),
    )(W_loc, X, bias)


# speedup vs baseline: 3.5898x; 1.0021x over previous
import jax
import jax.numpy as jnp
from jax.experimental import pallas as pl
from jax.experimental.pallas import tpu as pltpu

B, T, C = 4, 2048, 768
T_OUT = 2048
BC = 256


def _body(w_ref, x_ref, b_ref, o_ref, wt_ref):
    b_id = pl.program_id(0)
    c_id = pl.program_id(1)

    @pl.when(jnp.logical_and(b_id == 0, c_id == 0))
    def _():
        wt_ref[...] = w_ref[...].T.astype(jnp.bfloat16)

    x = x_ref[0]
    bias = b_ref[...]

    acc = jax.lax.dot_general(
        wt_ref[...], x.astype(jnp.bfloat16), (((1,), (0,)), ((), ())),
        preferred_element_type=jnp.float32,
    )
    d = jnp.tanh(acc + bias)

    neg = d < 0.0
    sat = d >= 1.0
    w1 = d - jnp.floor(d)
    xm1 = jnp.concatenate([x[:1], x[:-1]], axis=0)
    xp1 = jnp.concatenate([x[1:], x[-1:]], axis=0)
    va = jnp.where(neg, xm1, jnp.where(sat, xp1, x))
    vb = jnp.where(neg, x, xp1)
    out = va + w1 * (vb - va)

    zero = jnp.zeros((1, BC), jnp.float32)
    r0 = jnp.where(neg[:1], zero, out[:1])
    rl = jnp.where(neg[-1:], out[-1:], zero)
    rp = jnp.where(sat[T - 2:T - 1], zero, out[T - 2:T - 1])
    o_ref[0] = jnp.concatenate([r0, out[1:T - 2], rp, rl], axis=0)


@jax.jit
def kernel(X, W_loc, b_loc):
    bias = b_loc.reshape(T_OUT, 1)
    grid = (B, C // BC)
    return pl.pallas_call(
        _body,
        grid=grid,
        in_specs=[
            pl.BlockSpec((T, T_OUT), lambda b, c: (0, 0)),
            pl.BlockSpec((1, T, BC), lambda b, c: (b, 0, c)),
            pl.BlockSpec((T_OUT, 1), lambda b, c: (0, 0)),
        ],
        out_specs=pl.BlockSpec((1, T_OUT, BC), lambda b, c: (b, 0, c)),
        out_shape=jax.ShapeDtypeStruct((B, T_OUT, C), jnp.float32),
        scratch_shapes=[pltpu.VMEM((T_OUT, T), jnp.bfloat16)],
        compiler_params=pltpu.CompilerParams(
            dimension_semantics=("arbitrary", "arbitrary"),
        ),
    )(W_loc, X, bias)
